# baseline (speedup 1.0000x reference)
import jax
import jax.numpy as jnp
from jax.experimental import pallas as pl

_B = 4096
_S = 4096
_F = 2
_T = 64
_D_OUT = 1
_TAIL = 8


def _hijack_tail_kernel(x_ref, inds_ref, tokens_ref, w_ref, out_ref):
    xl = x_ref[:, _TAIL - 1, :]
    c0 = xl[:, 0:1]
    c1 = xl[:, 1:2]

    inds = inds_ref[...]
    iota = jax.lax.broadcasted_iota(jnp.int32, (1, _T), 1)
    hit = inds == (_S - 1)
    t_sel = jnp.max(jnp.where(hit, iota, -1))
    onehot = (iota == t_sel).astype(jnp.float32)
    repl = jnp.sum(tokens_ref[...] * onehot, axis=1, keepdims=True)
    c1 = jnp.where(t_sel >= 0, repl, c1)

    out_ref[...] = c0 * w_ref[0, 0] + c1 * w_ref[1, 0]


def kernel(x, inds, tokens, W):
    inds2 = inds.reshape(1, _T).astype(jnp.int32)
    return pl.pallas_call(
        _hijack_tail_kernel,
        grid=(1,),
        in_specs=[
            pl.BlockSpec((_B, _TAIL, _F), lambda i: (0, _S // _TAIL - 1, 0)),
            pl.BlockSpec((1, _T), lambda i: (0, 0)),
            pl.BlockSpec((_B, _T), lambda i: (0, 0)),
            pl.BlockSpec((_F, _D_OUT), lambda i: (0, 0)),
        ],
        out_specs=pl.BlockSpec((_B, _D_OUT), lambda i: (0, 0)),
        out_shape=jax.ShapeDtypeStruct((_B, _D_OUT), jnp.float32),
    )(x, inds2, tokens, W)


# speedup vs baseline: 9.5118x; 9.5118x over previous
import jax
import jax.numpy as jnp
from jax.experimental import pallas as pl

_B = 4096
_S = 4096
_F = 2
_T = 64
_D_OUT = 1
_TAIL = 8


def _hijack_tail_kernel(x_ref, inds_ref, tokens_ref, w_ref, out_ref):
    xt = x_ref[...]
    c0 = xt[:, 126:127]
    c1 = xt[:, 127:128]

    inds = inds_ref[...]
    iota = jax.lax.broadcasted_iota(jnp.int32, (1, _T), 1)
    hit = inds == (_S - 1)
    t_sel = jnp.max(jnp.where(hit, iota, -1))
    onehot = (iota == t_sel).astype(jnp.float32)
    repl = jnp.sum(tokens_ref[...] * onehot, axis=1, keepdims=True)
    c1 = jnp.where(t_sel >= 0, repl, c1)

    out_ref[...] = c0 * w_ref[0, 0] + c1 * w_ref[1, 0]


def kernel(x, inds, tokens, W):
    x2 = x.reshape(_B, _S * _F)
    inds2 = inds.reshape(1, _T).astype(jnp.int32)
    return pl.pallas_call(
        _hijack_tail_kernel,
        grid=(1,),
        in_specs=[
            pl.BlockSpec((_B, 128), lambda i: (0, _S * _F // 128 - 1)),
            pl.BlockSpec((1, _T), lambda i: (0, 0)),
            pl.BlockSpec((_B, _T), lambda i: (0, 0)),
            pl.BlockSpec((_F, _D_OUT), lambda i: (0, 0)),
        ],
        out_specs=pl.BlockSpec((_B, _D_OUT), lambda i: (0, 0)),
        out_shape=jax.ShapeDtypeStruct((_B, _D_OUT), jnp.float32),
    )(x2, inds2, tokens, W)
